# baseline (speedup 1.0000x reference)
import jax
import jax.numpy as jnp
from jax import lax
from jax.experimental import pallas as pl
from jax.experimental.pallas import tpu as pltpu
from jax.experimental.pallas import tpu_sc as plsc

_D = 16
_L = 20
_LP = 32
_C = 4
_CP = 16
_B = 16384
_NN = 999_999
_NCAT = 10_000
_NC = 2
_NS = 16
_NW = _NC * _NS
_SPW = _B // _NW
_CHUNK = 128
_NCH = _SPW // _CHUNK
_GRP = _CHUNK // 16


def _splat_i32(x):
    return jnp.full((16,), x, dtype=jnp.int32)


def _sqrt16(x):
    xc = jnp.maximum(x, 1e-30)
    i = lax.bitcast_convert_type(xc, jnp.int32)
    y = lax.bitcast_convert_type(jnp.int32(0x5F3759DF) - (i >> 1), jnp.float32)
    for _ in range(3):
        y = y * (1.5 - 0.5 * xc * y * y)
    return xc * y


def _softplus_neg(t):
    u = jnp.exp(-t)
    s = u / (u + 2.0)
    s2 = s * s
    return 2.0 * s * (1.0 + s2 * (1.0 / 3.0 + s2 * (0.2 + s2 * (1.0 / 7.0 + s2 * (1.0 / 9.0)))))


def _body(cen_hbm, ctx_hbm, item_hbm, catemb_hbm, nw_hbm,
          path_hbm, cat_hbm,
          out_hbm,
          cen_idx, ctx_idx, center_v, path_v, cat_v,
          w16, ce16, ct_all, out_stage, sem):
    wid = lax.axis_index("s") * _NC + lax.axis_index("c")
    iota = lax.iota(jnp.int32, 16)
    zf = jnp.zeros((16,), jnp.float32)

    def chunk_body(ch, acc):
        acc_bce, acc_pm, acc_per, acc_valid = acc
        base = wid * _SPW + ch * _CHUNK
        pltpu.sync_copy(cen_hbm.at[pl.ds(base, _CHUNK)], cen_idx)
        pltpu.sync_copy(ctx_hbm.at[pl.ds(base, _CHUNK)], ctx_idx)
        pltpu.async_copy(item_hbm.at[cen_idx], center_v, sem).wait()
        pltpu.async_copy(path_hbm.at[ctx_idx], path_v, sem).wait()
        pltpu.async_copy(cat_hbm.at[cen_idx], cat_v, sem).wait()

        for g in range(_GRP):
            rows = g * 16 + iota
            for d_ in range(_D):
                ct_all[g * 16 + d_] = plsc.load_gather(
                    center_v, [rows, _splat_i32(d_)])

        def l_body(l, lacc):
            l_bce, l_pm = lacc
            lsplat = iota * 0 + l
            for g in range(_GRP):
                rows = g * 16 + iota
                w = plsc.load_gather(path_v, [rows, lsplat])
                nid = jnp.minimum(w & 0xFFFFF, _NN - 1)
                pltpu.async_copy(nw_hbm.at[nid], w16, sem).wait()
                z = zf
                for d_ in range(_D):
                    z = z + (plsc.load_gather(w16, [iota, _splat_i32(d_)])
                             * ct_all[g * 16 + d_])
                y = ((w >> 20) & 1).astype(jnp.float32)
                pm = ((w >> 21) & 1).astype(jnp.float32)
                bce = jnp.maximum(z, 0.0) - z * y + _softplus_neg(jnp.abs(z))
                l_bce = l_bce + bce * pm
                l_pm = l_pm + pm
            return l_bce, l_pm

        acc_bce, acc_pm = lax.fori_loop(0, _L, l_body, (acc_bce, acc_pm))

        for g in range(_GRP):
            rows = g * 16 + iota
            n2c = zf
            for d_ in range(_D):
                cd = ct_all[g * 16 + d_]
                n2c = n2c + cd * cd
            cn = jnp.maximum(_sqrt16(n2c), 1e-8)
            per = zf
            cnt = zf
            for c_ in range(_C):
                cid = plsc.load_gather(cat_v, [rows, _splat_i32(c_)])
                cid = jnp.minimum(jnp.maximum(cid, 0), _NCAT - 1)
                pltpu.async_copy(catemb_hbm.at[cid], ce16, sem).wait()
                dotc = zf
                n2e = zf
                for d_ in range(_D):
                    ed = plsc.load_gather(ce16, [iota, _splat_i32(d_)])
                    dotc = dotc + ed * ct_all[g * 16 + d_]
                    n2e = n2e + ed * ed
                en = jnp.maximum(_sqrt16(n2e), 1e-8)
                cos = dotc / (cn * en)
                cm = plsc.load_gather(
                    cat_v, [rows, _splat_i32(_C + c_)]).astype(jnp.float32)
                per = per + (1.0 - cos) * cm
                cnt = cnt + cm
            has = cnt > 0.0
            acc_per = acc_per + jnp.where(has, per / jnp.maximum(cnt, 1.0), 0.0)
            acc_valid = acc_valid + jnp.where(has, 1.0, 0.0)

        return acc_bce, acc_pm, acc_per, acc_valid

    acc = lax.fori_loop(0, _NCH, chunk_body, (zf, zf, zf, zf))
    out_stage[pl.ds(0, 16)] = acc[0]
    out_stage[pl.ds(16, 16)] = acc[1]
    out_stage[pl.ds(32, 16)] = acc[2]
    out_stage[pl.ds(48, 16)] = acc[3]
    pltpu.sync_copy(out_stage, out_hbm.at[wid])


_sc_call = pl.kernel(
    _body,
    out_type=jax.ShapeDtypeStruct((_NW, 64), jnp.float32),
    mesh=plsc.VectorSubcoreMesh(core_axis_name="c", subcore_axis_name="s"),
    compiler_params=pltpu.CompilerParams(needs_layout_passes=False,
                                         use_tc_tiling_on_sc=False),
    scratch_types=[
        pltpu.VMEM((_CHUNK,), jnp.int32),
        pltpu.VMEM((_CHUNK,), jnp.int32),
        pltpu.VMEM((_CHUNK, _D), jnp.float32),
        pltpu.VMEM((_CHUNK, _LP), jnp.int32),
        pltpu.VMEM((_CHUNK, _CP), jnp.int32),
        pltpu.VMEM((16, _D), jnp.float32),
        pltpu.VMEM((16, _D), jnp.float32),
        pltpu.VMEM((_GRP * _D, 16), jnp.float32),
        pltpu.VMEM((64,), jnp.float32),
        pltpu.SemaphoreType.DMA,
    ],
)


def kernel(center_ids, context_ids, item_emb, category_emb, node_weights,
           codes_tbl, node_ids_tbl, path_mask_tbl, cat_ids_tbl, cat_mask_tbl):
    path_pack = (node_ids_tbl
                 | (codes_tbl << 20)
                 | (path_mask_tbl.astype(jnp.int32) << 21))
    path_pack = jnp.pad(path_pack, ((0, 0), (0, _LP - _L)))
    n_items = cat_ids_tbl.shape[0]
    cat_pack = jnp.concatenate(
        [cat_ids_tbl, cat_mask_tbl.astype(jnp.int32),
         jnp.zeros((n_items, _CP - 2 * _C), jnp.int32)], axis=1)
    parts = _sc_call(center_ids, context_ids, item_emb, category_emb,
                     node_weights, path_pack, cat_pack)
    p = parts.reshape(_NW, 4, 16).sum(axis=(0, 2))
    hs_loss = p[0] / p[1]
    cat_loss = jnp.where(p[3] > 0, p[2] / jnp.maximum(p[3], 1.0), 0.0)
    return hs_loss + 0.1 * cat_loss


# speedup vs baseline: 1.0971x; 1.0971x over previous
import jax
import jax.numpy as jnp
from jax import lax
from jax.experimental import pallas as pl
from jax.experimental.pallas import tpu as pltpu
from jax.experimental.pallas import tpu_sc as plsc

_D = 16
_L = 20
_LP = 32
_C = 4
_CP = 16
_B = 16384
_NN = 999_999
_NCAT = 10_000
_NC = 2
_NS = 16
_NW = _NC * _NS
_SPW = _B // _NW
_CHUNK = 128
_NCH = _SPW // _CHUNK
_GRP = _CHUNK // 16


def _splat_i32(x):
    return jnp.full((16,), x, dtype=jnp.int32)


def _sqrt16(x):
    xc = jnp.maximum(x, 1e-30)
    i = lax.bitcast_convert_type(xc, jnp.int32)
    y = lax.bitcast_convert_type(jnp.int32(0x5F3759DF) - (i >> 1), jnp.float32)
    for _ in range(3):
        y = y * (1.5 - 0.5 * xc * y * y)
    return xc * y


def _softplus_neg(t):
    u = jnp.exp(-t)
    s = u / (u + 2.0)
    s2 = s * s
    return 2.0 * s * (1.0 + s2 * (1.0 / 3.0 + s2 * (0.2 + s2 * (1.0 / 7.0 + s2 * (1.0 / 9.0)))))


def _body(cen_hbm, ctx_hbm, item_hbm, catemb_hbm, nw_hbm,
          path_hbm, cat_hbm,
          out_hbm,
          cen_idx, ctx_idx, center_v, path_v, cat_v,
          idx_l, wl_v, ce0, ce1, ce2, ce3, ct_all, out_stage, sem):
    wid = lax.axis_index("s") * _NC + lax.axis_index("c")
    iota = lax.iota(jnp.int32, 16)
    zf = jnp.zeros((16,), jnp.float32)

    def chunk_body(ch, acc):
        acc_bce, acc_pm, acc_per, acc_valid = acc
        base = wid * _SPW + ch * _CHUNK
        pltpu.sync_copy(cen_hbm.at[pl.ds(base, _CHUNK)], cen_idx)
        pltpu.sync_copy(ctx_hbm.at[pl.ds(base, _CHUNK)], ctx_idx)
        pltpu.async_copy(item_hbm.at[cen_idx], center_v, sem).wait()
        pltpu.async_copy(path_hbm.at[ctx_idx], path_v, sem).wait()
        pltpu.async_copy(cat_hbm.at[cen_idx], cat_v, sem).wait()

        for g in range(_GRP):
            rows = g * 16 + iota
            for d_ in range(_D):
                ct_all[g * 16 + d_] = plsc.load_gather(
                    center_v, [rows, _splat_i32(d_)])

        def l_body(l, lacc):
            l_bce, l_pm = lacc
            lsplat = iota * 0 + l
            for v in range(_GRP):
                w = plsc.load_gather(path_v, [v * 16 + iota, lsplat])
                idx_l[pl.ds(v * 16, 16)] = jnp.minimum(w & 0xFFFFF, _NN - 1)
            pltpu.async_copy(nw_hbm.at[idx_l], wl_v, sem).wait()
            for g in range(_GRP):
                rows = g * 16 + iota
                w = plsc.load_gather(path_v, [rows, lsplat])
                z = zf
                for d_ in range(_D):
                    z = z + (plsc.load_gather(wl_v, [rows, _splat_i32(d_)])
                             * ct_all[g * 16 + d_])
                y = ((w >> 20) & 1).astype(jnp.float32)
                pm = ((w >> 21) & 1).astype(jnp.float32)
                bce = jnp.maximum(z, 0.0) - z * y + _softplus_neg(jnp.abs(z))
                l_bce = l_bce + bce * pm
                l_pm = l_pm + pm
            return l_bce, l_pm

        acc_bce, acc_pm = lax.fori_loop(0, _L, l_body, (acc_bce, acc_pm))

        ce_bufs = (ce0, ce1, ce2, ce3)
        for c_ in range(_C):
            csplat = _splat_i32(c_)
            for v in range(_GRP):
                cid = plsc.load_gather(cat_v, [v * 16 + iota, csplat])
                idx_l[pl.ds(v * 16, 16)] = jnp.minimum(
                    jnp.maximum(cid, 0), _NCAT - 1)
            pltpu.async_copy(catemb_hbm.at[idx_l], ce_bufs[c_], sem).wait()

        for g in range(_GRP):
            rows = g * 16 + iota
            n2c = zf
            for d_ in range(_D):
                cd = ct_all[g * 16 + d_]
                n2c = n2c + cd * cd
            cn = jnp.maximum(_sqrt16(n2c), 1e-8)
            per = zf
            cnt = zf
            for c_ in range(_C):
                dotc = zf
                n2e = zf
                for d_ in range(_D):
                    ed = plsc.load_gather(ce_bufs[c_], [rows, _splat_i32(d_)])
                    dotc = dotc + ed * ct_all[g * 16 + d_]
                    n2e = n2e + ed * ed
                en = jnp.maximum(_sqrt16(n2e), 1e-8)
                cos = dotc / (cn * en)
                cm = plsc.load_gather(
                    cat_v, [rows, _splat_i32(_C + c_)]).astype(jnp.float32)
                per = per + (1.0 - cos) * cm
                cnt = cnt + cm
            has = cnt > 0.0
            acc_per = acc_per + jnp.where(has, per / jnp.maximum(cnt, 1.0), 0.0)
            acc_valid = acc_valid + jnp.where(has, 1.0, 0.0)

        return acc_bce, acc_pm, acc_per, acc_valid

    acc = lax.fori_loop(0, _NCH, chunk_body, (zf, zf, zf, zf))
    out_stage[pl.ds(0, 16)] = acc[0]
    out_stage[pl.ds(16, 16)] = acc[1]
    out_stage[pl.ds(32, 16)] = acc[2]
    out_stage[pl.ds(48, 16)] = acc[3]
    pltpu.sync_copy(out_stage, out_hbm.at[wid])


_sc_call = pl.kernel(
    _body,
    out_type=jax.ShapeDtypeStruct((_NW, 64), jnp.float32),
    mesh=plsc.VectorSubcoreMesh(core_axis_name="c", subcore_axis_name="s"),
    compiler_params=pltpu.CompilerParams(needs_layout_passes=False,
                                         use_tc_tiling_on_sc=False),
    scratch_types=[
        pltpu.VMEM((_CHUNK,), jnp.int32),
        pltpu.VMEM((_CHUNK,), jnp.int32),
        pltpu.VMEM((_CHUNK, _D), jnp.float32),
        pltpu.VMEM((_CHUNK, _LP), jnp.int32),
        pltpu.VMEM((_CHUNK, _CP), jnp.int32),
        pltpu.VMEM((_CHUNK,), jnp.int32),
        pltpu.VMEM((_CHUNK, _D), jnp.float32),
        pltpu.VMEM((_CHUNK, _D), jnp.float32),
        pltpu.VMEM((_CHUNK, _D), jnp.float32),
        pltpu.VMEM((_CHUNK, _D), jnp.float32),
        pltpu.VMEM((_CHUNK, _D), jnp.float32),
        pltpu.VMEM((_GRP * _D, 16), jnp.float32),
        pltpu.VMEM((64,), jnp.float32),
        pltpu.SemaphoreType.DMA,
    ],
)


def kernel(center_ids, context_ids, item_emb, category_emb, node_weights,
           codes_tbl, node_ids_tbl, path_mask_tbl, cat_ids_tbl, cat_mask_tbl):
    path_pack = (node_ids_tbl
                 | (codes_tbl << 20)
                 | (path_mask_tbl.astype(jnp.int32) << 21))
    path_pack = jnp.pad(path_pack, ((0, 0), (0, _LP - _L)))
    n_items = cat_ids_tbl.shape[0]
    cat_pack = jnp.concatenate(
        [cat_ids_tbl, cat_mask_tbl.astype(jnp.int32),
         jnp.zeros((n_items, _CP - 2 * _C), jnp.int32)], axis=1)
    parts = _sc_call(center_ids, context_ids, item_emb, category_emb,
                     node_weights, path_pack, cat_pack)
    p = parts.reshape(_NW, 4, 16).sum(axis=(0, 2))
    hs_loss = p[0] / p[1]
    cat_loss = jnp.where(p[3] > 0, p[2] / jnp.maximum(p[3], 1.0), 0.0)
    return hs_loss + 0.1 * cat_loss
